# baseline (speedup 1.0000x reference)
import math

import jax
import jax.numpy as jnp
from jax import lax
from jax.experimental import pallas as pl
from jax.experimental.pallas import tpu as pltpu
from jax.experimental.pallas import tpu_sc as plsc

D_OUT = 128
SEQ = (4096, 50)
N_TOK = SEQ[0] * SEQ[1]
NH = N_TOK // 2
NC, NS, L = 2, 16, 16
NW = NC * NS
BPW = N_TOK // NW
CHUNK = 128
CAP = BPW + CHUNK
NPAD = NW * CHUNK
NSTG = N_TOK + NPAD
SCALE = math.sqrt(float(D_OUT))

C0_HI = 20000
C1_HI = 100000

B = 2048
GRID = NH // B


def _sc_body(ids_hbm, emb0_hbm, emb1_hbm, emb2_hbm, s1_hbm, s2_hbm,
             ids_v, p0_v, x0_v, p1_v, x1_v, p2_v, x2_v,
             idx_s, pos_s, r0_v, r1_v, r2_v, r0a_v, r0b_v, r2w_v, sem):
    wid = lax.axis_index("s") * NC + lax.axis_index("c")
    base = wid * BPW
    pltpu.sync_copy(ids_hbm.at[pl.ds(base, BPW)], ids_v)
    iota = lax.iota(jnp.int32, L)

    def zero_body(i, carry):
        r2w_v[i, pl.ds(32, L)] = jnp.zeros((L,), jnp.float32)
        r2w_v[i, pl.ds(48, L)] = jnp.zeros((L,), jnp.float32)
        return carry

    lax.fori_loop(0, CHUNK, zero_body, 0)

    dump_base = jnp.int32(N_TOK) + wid * CHUNK

    def init_body(i, carry):
        c = (i * L) % CHUNK
        dpos = dump_base + c + iota
        didx = c + iota
        for pv in (p0_v, p1_v, p2_v):
            pv[pl.ds(i * L, L)] = dpos
        for xv in (x0_v, x1_v, x2_v):
            xv[pl.ds(i * L, L)] = didx
        return carry

    lax.fori_loop(0, CAP // L, init_body, 0)

    trash = jnp.int32(CAP - L) + iota

    def scan_body(i, wps):
        w0, w1, w2 = wps
        v = ids_v[pl.ds(i * L, L)]
        r = jnp.int32(base) + i * L + iota
        q = r * 2 - jnp.where(r < NH, jnp.int32(0), jnp.int32(N_TOK - 1))
        m0 = v < C0_HI
        m1 = jnp.logical_and(v >= C0_HI, v < C1_HI)
        m2 = v >= C1_HI

        def emit(m, w, pv, xv, val):
            s = plsc.cumsum(jnp.where(m, jnp.int32(1), jnp.int32(0)))
            offs = jnp.where(m, w + s - 1, trash)
            plsc.store_scatter(pv, [offs], q)
            plsc.store_scatter(xv, [offs], val)
            return w + plsc.all_reduce_population_count(m)

        w0 = emit(m0, w0, p0_v, x0_v, v)
        w1 = emit(m1, w1, p1_v, x1_v, v - C0_HI)
        w2 = emit(m2, w2, p2_v, x2_v, v - C1_HI)
        return (w0, w1, w2)

    z = jnp.zeros((L,), jnp.int32)
    w0_v, w1_v, w2_v = lax.fori_loop(0, BPW // L, scan_body, (z, z, z))
    w0 = jnp.max(w0_v)
    w1 = jnp.max(w1_v)
    w2 = jnp.max(w2_v)

    def stage_chunk(c, pos_arr, idx_arr):
        o = c * CHUNK
        for k in range(CHUNK // L):
            idx_s[pl.ds(k * L, L)] = idx_arr[pl.ds(o + k * L, L)]
            pos_s[pl.ds(k * L, L)] = pos_arr[pl.ds(o + k * L, L)]

    def c1_chunk(c, carry):
        stage_chunk(c, p1_v, x1_v)
        pltpu.async_copy(emb1_hbm.at[idx_s], r1_v, sem).wait()
        pltpu.async_copy(r1_v, s1_hbm.at[pos_s], sem).wait()
        return carry

    lax.fori_loop(0, (w1 + CHUNK - 1) // CHUNK, c1_chunk, 0)

    def c2_chunk(c, carry):
        stage_chunk(c, p2_v, x2_v)
        pltpu.async_copy(emb2_hbm.at[idx_s], r2_v, sem).wait()

        def repack(rr, rc):
            r2w_v[rr, pl.ds(0, L)] = r2_v[rr, pl.ds(0, L)]
            r2w_v[rr, pl.ds(L, L)] = r2_v[rr, pl.ds(L, L)]
            return rc

        lax.fori_loop(0, CHUNK, repack, 0)
        pltpu.async_copy(r2w_v, s2_hbm.at[pos_s], sem).wait()
        return carry

    lax.fori_loop(0, (w2 + CHUNK - 1) // CHUNK, c2_chunk, 0)

    def c0_chunk(c, carry):
        stage_chunk(c, p0_v, x0_v)
        pltpu.async_copy(emb0_hbm.at[idx_s], r0_v, sem).wait()

        def repack(rr, rc):
            for k in range(4):
                r0a_v[rr, pl.ds(k * L, L)] = r0_v[rr, pl.ds(k * L, L)]
            for k in range(4):
                r0b_v[rr, pl.ds(k * L, L)] = r0_v[rr, pl.ds(64 + k * L, L)]
            return rc

        lax.fori_loop(0, CHUNK, repack, 0)
        cpa = pltpu.async_copy(r0a_v, s1_hbm.at[pos_s], sem)
        cpb = pltpu.async_copy(r0b_v, s2_hbm.at[pos_s], sem)
        cpa.wait()
        cpb.wait()
        return carry

    lax.fori_loop(0, (w0 + CHUNK - 1) // CHUNK, c0_chunk, 0)


def _sc_stage(ids, emb0, emb1, emb2):
    mesh = plsc.VectorSubcoreMesh(
        core_axis_name="c", subcore_axis_name="s",
        num_cores=NC, num_subcores=NS)
    call = pl.kernel(
        _sc_body,
        out_type=[
            jax.ShapeDtypeStruct((NSTG, 64), jnp.float32),
            jax.ShapeDtypeStruct((NSTG, 64), jnp.float32),
        ],
        mesh=mesh,
        compiler_params=pltpu.CompilerParams(
            use_tc_tiling_on_sc=False, needs_layout_passes=False),
        scratch_types=[
            pltpu.VMEM((BPW,), jnp.int32),
            pltpu.VMEM((CAP,), jnp.int32),
            pltpu.VMEM((CAP,), jnp.int32),
            pltpu.VMEM((CAP,), jnp.int32),
            pltpu.VMEM((CAP,), jnp.int32),
            pltpu.VMEM((CAP,), jnp.int32),
            pltpu.VMEM((CAP,), jnp.int32),
            pltpu.VMEM((CHUNK,), jnp.int32),
            pltpu.VMEM((CHUNK,), jnp.int32),
            pltpu.VMEM((CHUNK, 128), jnp.float32),
            pltpu.VMEM((CHUNK, 64), jnp.float32),
            pltpu.VMEM((CHUNK, 32), jnp.float32),
            pltpu.VMEM((CHUNK, 64), jnp.float32),
            pltpu.VMEM((CHUNK, 64), jnp.float32),
            pltpu.VMEM((CHUNK, 64), jnp.float32),
            pltpu.SemaphoreType.DMA,
        ],
    )
    return call(ids, emb0, emb1, emb2)


def _tc_body(ids_ref, s1_ref, s2_ref, w1_ref, w2_ref, o_ref):
    b1 = s1_ref[...]
    b2 = s2_ref[...]
    w1t = w1_ref[...]
    w2t = w2_ref[...]
    G = B // 128
    rows = lax.broadcasted_iota(jnp.int32, (B, G), 0) // 128
    cols = lax.broadcasted_iota(jnp.int32, (B, G), 1)
    oh = (rows == cols).astype(jnp.float32)
    lane = lax.broadcasted_iota(jnp.int32, (B, 128), 1)
    pick = (lane == lax.broadcasted_iota(jnp.int32, (B, 128), 0) % 128)
    pickf = pick.astype(jnp.float32)
    outs = []
    for h in range(2):
        sl = slice(64 * h, 64 * h + 64)
        p1 = jnp.dot(b1[:, sl], w1t, preferred_element_type=jnp.float32)
        p2 = jnp.dot(b2[:, sl], w2t, preferred_element_type=jnp.float32)
        raw = jnp.concatenate([b1[:, sl], b2[:, sl]], axis=1)
        idsf = ids_ref[h].astype(jnp.float32)
        spread = jnp.dot(oh, idsf, preferred_element_type=jnp.float32,
                         precision=lax.Precision.HIGHEST)
        idcol = jnp.sum(spread * pickf, axis=1, keepdims=True)
        m0 = idcol < float(C0_HI)
        m1 = idcol < float(C1_HI)
        outs.append(jnp.where(m0, raw, jnp.where(m1, p1, p2)) * SCALE)
    o_ref[...] = jnp.stack(outs, axis=0)


def _tc_combine(ids3, s1v, s2v, w1t, w2te):
    return pl.pallas_call(
        _tc_body,
        grid=(GRID,),
        in_specs=[
            pl.BlockSpec((2, B // 128, 128), lambda j: (0, j, 0)),
            pl.BlockSpec((B, 128), lambda j: (j, 0)),
            pl.BlockSpec((B, 128), lambda j: (j, 0)),
            pl.BlockSpec((64, 128), lambda j: (0, 0)),
            pl.BlockSpec((64, 128), lambda j: (0, 0)),
        ],
        out_specs=pl.BlockSpec((2, B, 128), lambda j: (0, j, 0)),
        out_shape=jax.ShapeDtypeStruct((2, NH, 128), jnp.float32),
    )(ids3, s1v, s2v, w1t, w2te)


def kernel(input_ids, emb0, emb1, emb2, W1, W2):
    ids = input_ids.T.reshape(-1)
    s1, s2 = _sc_stage(ids, emb0, emb1, emb2)
    s1v = s1.reshape(NSTG // 2, 128)
    s2v = s2.reshape(NSTG // 2, 128)
    ids3 = ids.reshape(2, NH // 128, 128)
    w2te = jnp.concatenate(
        [W2.T, jnp.zeros((32, D_OUT), jnp.float32)], axis=0)
    out = _tc_combine(ids3, s1v, s2v, W1.T, w2te)
    return out.reshape(SEQ[1], SEQ[0], D_OUT).transpose(1, 0, 2)


# speedup vs baseline: 1.0474x; 1.0474x over previous
import math

import jax
import jax.numpy as jnp
from jax import lax
from jax.experimental import pallas as pl
from jax.experimental.pallas import tpu as pltpu
from jax.experimental.pallas import tpu_sc as plsc

D_OUT = 128
SEQ = (4096, 50)
N_TOK = SEQ[0] * SEQ[1]
NH = N_TOK // 2
NC, NS, L = 2, 16, 16
NW = NC * NS
BPW = N_TOK // NW
CHUNK = 128
CAP = BPW + CHUNK
NPAD = NW * CHUNK
NSTG = N_TOK + NPAD
SCALE = math.sqrt(float(D_OUT))

C0_HI = 20000
C1_HI = 100000

B = 2048
GRID = NH // B


def _sc_body(ids_hbm, emb0_hbm, emb1_hbm, emb2_hbm, s1_hbm, s2_hbm,
             ids_v, p0_v, x0_v, p1_v, x1_v, p2_v, x2_v,
             idx_s, pos_s, r0_v, r1_v, r2_v, r0a_v, r0b_v, r2w_v, sem):
    wid = lax.axis_index("s") * NC + lax.axis_index("c")
    base = wid * BPW
    pltpu.sync_copy(ids_hbm.at[pl.ds(base, BPW)], ids_v)
    iota = lax.iota(jnp.int32, L)

    def zero_body(i, carry):
        r2w_v[i, pl.ds(32, L)] = jnp.zeros((L,), jnp.float32)
        r2w_v[i, pl.ds(48, L)] = jnp.zeros((L,), jnp.float32)
        return carry

    lax.fori_loop(0, CHUNK, zero_body, 0)

    dump_base = jnp.int32(N_TOK) + wid * CHUNK

    def init_body(i, carry):
        c = (i * L) % CHUNK
        dpos = dump_base + c + iota
        didx = c + iota
        for pv in (p0_v, p1_v, p2_v):
            pv[pl.ds(i * L, L)] = dpos
        for xv in (x0_v, x1_v, x2_v):
            xv[pl.ds(i * L, L)] = didx
        return carry

    lax.fori_loop(0, CAP // L, init_body, 0)

    trash = jnp.int32(CAP - L) + iota

    def scan_body(i, wps):
        w0, w1, w2 = wps
        v = ids_v[pl.ds(i * L, L)]
        r = jnp.int32(base) + i * L + iota
        q = r * 2 - jnp.where(r < NH, jnp.int32(0), jnp.int32(N_TOK - 1))
        m0 = v < C0_HI
        m1 = jnp.logical_and(v >= C0_HI, v < C1_HI)
        m2 = v >= C1_HI

        def emit(m, w, pv, xv, val):
            s = plsc.cumsum(jnp.where(m, jnp.int32(1), jnp.int32(0)))
            offs = jnp.where(m, w + s - 1, trash)
            plsc.store_scatter(pv, [offs], q)
            plsc.store_scatter(xv, [offs], val)
            return w + plsc.all_reduce_population_count(m)

        w0 = emit(m0, w0, p0_v, x0_v, v)
        w1 = emit(m1, w1, p1_v, x1_v, v - C0_HI)
        w2 = emit(m2, w2, p2_v, x2_v, v - C1_HI)
        return (w0, w1, w2)

    z = jnp.zeros((L,), jnp.int32)
    w0_v, w1_v, w2_v = lax.fori_loop(0, BPW // L, scan_body, (z, z, z))
    w0 = jnp.max(w0_v)
    w1 = jnp.max(w1_v)
    w2 = jnp.max(w2_v)

    def stage_chunk(c, pos_arr, idx_arr):
        o = c * CHUNK
        for k in range(CHUNK // L):
            idx_s[pl.ds(k * L, L)] = idx_arr[pl.ds(o + k * L, L)]
            pos_s[pl.ds(k * L, L)] = pos_arr[pl.ds(o + k * L, L)]

    def c1_chunk(c, carry):
        stage_chunk(c, p1_v, x1_v)
        pltpu.async_copy(emb1_hbm.at[idx_s], r1_v, sem).wait()
        pltpu.async_copy(r1_v, s1_hbm.at[pos_s], sem).wait()
        return carry

    lax.fori_loop(0, (w1 + CHUNK - 1) // CHUNK, c1_chunk, 0)

    def c2_chunk(c, carry):
        stage_chunk(c, p2_v, x2_v)
        pltpu.async_copy(emb2_hbm.at[idx_s], r2_v, sem).wait()

        def repack(rr, rc):
            r2w_v[rr, pl.ds(0, L)] = r2_v[rr, pl.ds(0, L)]
            r2w_v[rr, pl.ds(L, L)] = r2_v[rr, pl.ds(L, L)]
            return rc

        lax.fori_loop(0, CHUNK, repack, 0)
        pltpu.async_copy(r2w_v, s2_hbm.at[pos_s], sem).wait()
        return carry

    lax.fori_loop(0, (w2 + CHUNK - 1) // CHUNK, c2_chunk, 0)

    def c0_chunk(c, carry):
        stage_chunk(c, p0_v, x0_v)
        pltpu.async_copy(emb0_hbm.at[idx_s], r0_v, sem).wait()

        def repack(rr, rc):
            for k in range(4):
                r0a_v[rr, pl.ds(k * L, L)] = r0_v[rr, pl.ds(k * L, L)]
            for k in range(4):
                r0b_v[rr, pl.ds(k * L, L)] = r0_v[rr, pl.ds(64 + k * L, L)]
            return rc

        lax.fori_loop(0, CHUNK, repack, 0)
        cpa = pltpu.async_copy(r0a_v, s1_hbm.at[pos_s], sem)
        cpb = pltpu.async_copy(r0b_v, s2_hbm.at[pos_s], sem)
        cpa.wait()
        cpb.wait()
        return carry

    lax.fori_loop(0, (w0 + CHUNK - 1) // CHUNK, c0_chunk, 0)


def _sc_stage(ids, emb0, emb1, emb2):
    mesh = plsc.VectorSubcoreMesh(
        core_axis_name="c", subcore_axis_name="s",
        num_cores=NC, num_subcores=NS)
    call = pl.kernel(
        _sc_body,
        out_type=[
            jax.ShapeDtypeStruct((NSTG, 64), jnp.float32),
            jax.ShapeDtypeStruct((NSTG, 64), jnp.float32),
        ],
        mesh=mesh,
        compiler_params=pltpu.CompilerParams(
            use_tc_tiling_on_sc=False, needs_layout_passes=False),
        scratch_types=[
            pltpu.VMEM((BPW,), jnp.int32),
            pltpu.VMEM((CAP,), jnp.int32),
            pltpu.VMEM((CAP,), jnp.int32),
            pltpu.VMEM((CAP,), jnp.int32),
            pltpu.VMEM((CAP,), jnp.int32),
            pltpu.VMEM((CAP,), jnp.int32),
            pltpu.VMEM((CAP,), jnp.int32),
            pltpu.VMEM((CHUNK,), jnp.int32),
            pltpu.VMEM((CHUNK,), jnp.int32),
            pltpu.VMEM((CHUNK, 128), jnp.float32),
            pltpu.VMEM((CHUNK, 64), jnp.float32),
            pltpu.VMEM((CHUNK, 32), jnp.float32),
            pltpu.VMEM((CHUNK, 64), jnp.float32),
            pltpu.VMEM((CHUNK, 64), jnp.float32),
            pltpu.VMEM((CHUNK, 64), jnp.float32),
            pltpu.SemaphoreType.DMA,
        ],
    )
    return call(ids, emb0, emb1, emb2)


def _tc_body(ids_ref, s1_ref, s2_ref, w1_ref, w2_ref, o_ref):
    b1 = s1_ref[...]
    b2 = s2_ref[...]
    w1t = w1_ref[...]
    w2t = w2_ref[...]
    G = B // 128
    rows = lax.broadcasted_iota(jnp.int32, (B, G), 0) // 128
    cols = lax.broadcasted_iota(jnp.int32, (B, G), 1)
    oh = (rows == cols).astype(jnp.float32)
    lane = lax.broadcasted_iota(jnp.int32, (B, 128), 1)
    pick = (lane == lax.broadcasted_iota(jnp.int32, (B, 128), 0) % 128)
    pickf = pick.astype(jnp.float32)
    outs = []
    for h in range(2):
        sl = slice(64 * h, 64 * h + 64)
        p1 = jnp.dot(b1[:, sl], w1t, preferred_element_type=jnp.float32)
        p2 = jnp.dot(b2[:, sl], w2t, preferred_element_type=jnp.float32)
        raw = jnp.concatenate([b1[:, sl], b2[:, sl]], axis=1)
        ids = ids_ref[h]
        f0 = (ids < C0_HI).astype(jnp.float32)
        f1 = (ids < C1_HI).astype(jnp.float32)
        s0 = jnp.dot(oh, f0, preferred_element_type=jnp.float32)
        s1 = jnp.dot(oh, f1, preferred_element_type=jnp.float32)
        m0 = jnp.sum(s0 * pickf, axis=1, keepdims=True) > 0.5
        m1 = jnp.sum(s1 * pickf, axis=1, keepdims=True) > 0.5
        outs.append(jnp.where(m0, raw, jnp.where(m1, p1, p2)) * SCALE)
    o_ref[...] = jnp.stack(outs, axis=0)


def _tc_combine(ids3, s1v, s2v, w1t, w2te):
    return pl.pallas_call(
        _tc_body,
        grid=(GRID,),
        in_specs=[
            pl.BlockSpec((2, B // 128, 128), lambda j: (0, j, 0)),
            pl.BlockSpec((B, 128), lambda j: (j, 0)),
            pl.BlockSpec((B, 128), lambda j: (j, 0)),
            pl.BlockSpec((64, 128), lambda j: (0, 0)),
            pl.BlockSpec((64, 128), lambda j: (0, 0)),
        ],
        out_specs=pl.BlockSpec((2, B, 128), lambda j: (0, j, 0)),
        out_shape=jax.ShapeDtypeStruct((2, NH, 128), jnp.float32),
    )(ids3, s1v, s2v, w1t, w2te)


def kernel(input_ids, emb0, emb1, emb2, W1, W2):
    ids = input_ids.T.reshape(-1)
    s1, s2 = _sc_stage(ids, emb0, emb1, emb2)
    s1v = s1.reshape(NSTG // 2, 128)
    s2v = s2.reshape(NSTG // 2, 128)
    ids3 = ids.reshape(2, NH // 128, 128)
    w2te = jnp.concatenate(
        [W2.T, jnp.zeros((32, D_OUT), jnp.float32)], axis=0)
    out = _tc_combine(ids3, s1v, s2v, W1.T, w2te)
    return out.reshape(SEQ[1], SEQ[0], D_OUT).transpose(1, 0, 2)


# speedup vs baseline: 1.0715x; 1.0230x over previous
import math

import jax
import jax.numpy as jnp
from jax import lax
from jax.experimental import pallas as pl
from jax.experimental.pallas import tpu as pltpu
from jax.experimental.pallas import tpu_sc as plsc

D_OUT = 128
SEQ = (4096, 50)
N_TOK = SEQ[0] * SEQ[1]
NH = N_TOK // 2
NC, NS, L = 2, 16, 16
NW = NC * NS
BPW = N_TOK // NW
CHUNK = 128
CAP = BPW + CHUNK
NPAD = NW * CHUNK
NSTG = N_TOK + NPAD
SCALE = math.sqrt(float(D_OUT))

C0_HI = 20000
C1_HI = 100000

B = 2048
GRID = NH // B


def _sc_body(ids_hbm, emb0_hbm, emb1_hbm, emb2_hbm, s1_hbm, s2_hbm,
             ids_v, p0_v, x0_v, p1_v, x1_v, p2_v, x2_v,
             idx2_s, pos2_s, r0_v, r1_v, r2_v, r0a_v, r0b_v, r1w_v, r2w_v,
             sem, sem_s):
    wid = lax.axis_index("s") * NC + lax.axis_index("c")
    base = wid * BPW
    pltpu.sync_copy(ids_hbm.at[pl.ds(base, BPW)], ids_v)
    iota = lax.iota(jnp.int32, L)

    def zero_body(i, carry):
        r2w_v[i, pl.ds(32, L)] = jnp.zeros((L,), jnp.float32)
        r2w_v[i, pl.ds(48, L)] = jnp.zeros((L,), jnp.float32)
        return carry

    lax.fori_loop(0, CHUNK, zero_body, 0)

    dump_base = jnp.int32(N_TOK) + wid * CHUNK

    def init_body(i, carry):
        c = (i * L) % CHUNK
        dpos = dump_base + c + iota
        didx = c + iota
        for pv in (p0_v, p1_v, p2_v):
            pv[pl.ds(i * L, L)] = dpos
        for xv in (x0_v, x1_v, x2_v):
            xv[pl.ds(i * L, L)] = didx
        return carry

    lax.fori_loop(0, CAP // L, init_body, 0)

    trash = jnp.int32(CAP - L) + iota

    def scan_body(i, wps):
        w0, w1, w2 = wps
        v = ids_v[pl.ds(i * L, L)]
        r = jnp.int32(base) + i * L + iota
        q = r * 2 - jnp.where(r < NH, jnp.int32(0), jnp.int32(N_TOK - 1))
        m0 = v < C0_HI
        m1 = jnp.logical_and(v >= C0_HI, v < C1_HI)
        m2 = v >= C1_HI

        def emit(m, w, pv, xv, val):
            s = plsc.cumsum(jnp.where(m, jnp.int32(1), jnp.int32(0)))
            offs = jnp.where(m, w + s - 1, trash)
            plsc.store_scatter(pv, [offs], q)
            plsc.store_scatter(xv, [offs], val)
            return w + plsc.all_reduce_population_count(m)

        w0 = emit(m0, w0, p0_v, x0_v, v)
        w1 = emit(m1, w1, p1_v, x1_v, v - C0_HI)
        w2 = emit(m2, w2, p2_v, x2_v, v - C1_HI)
        return (w0, w1, w2)

    z = jnp.zeros((L,), jnp.int32)
    w0_v, w1_v, w2_v = lax.fori_loop(0, BPW // L, scan_body, (z, z, z))
    w0 = jnp.max(w0_v)
    w1 = jnp.max(w1_v)
    w2 = jnp.max(w2_v)

    def prestage(n, pos_arr, idx_arr):
        def body(c, carry):
            o = c * CHUNK
            for k in range(CHUNK // L):
                idx2_s[c, pl.ds(k * L, L)] = idx_arr[pl.ds(o + k * L, L)]
                pos2_s[c, pl.ds(k * L, L)] = pos_arr[pl.ds(o + k * L, L)]
            return carry

        lax.fori_loop(0, n, body, 0)

    def pipelined(n, table_hbm, row_buf, repack, scatters):
        def fire_gather(c):
            pltpu.async_copy(table_hbm.at[idx2_s.at[c]], row_buf, sem)

        def wait_gather():
            pltpu.make_async_copy(
                table_hbm.at[idx2_s.at[0]], row_buf, sem).wait()

        def body(c, carry):
            wait_gather()

            @pl.when(c > 0)
            def _():
                for src, dst in scatters:
                    pltpu.make_async_copy(
                        src, dst.at[pos2_s.at[0]], sem_s).wait()

            repack()

            @pl.when(c + 1 < n)
            def _():
                fire_gather(c + 1)

            for src, dst in scatters:
                pltpu.async_copy(src, dst.at[pos2_s.at[c]], sem_s)
            return carry

        @pl.when(n > 0)
        def _():
            fire_gather(jnp.int32(0))
            lax.fori_loop(0, n, body, 0)
            for src, dst in scatters:
                pltpu.make_async_copy(
                    src, dst.at[pos2_s.at[0]], sem_s).wait()

    def repack1():
        def body(rr, rc):
            for k in range(4):
                r1w_v[rr, pl.ds(k * L, L)] = r1_v[rr, pl.ds(k * L, L)]
            return rc

        lax.fori_loop(0, CHUNK, body, 0)

    n1 = (w1 + CHUNK - 1) // CHUNK
    prestage(n1, p1_v, x1_v)
    pipelined(n1, emb1_hbm, r1_v, repack1, [(r1w_v, s1_hbm)])

    def repack2():
        def body(rr, rc):
            r2w_v[rr, pl.ds(0, L)] = r2_v[rr, pl.ds(0, L)]
            r2w_v[rr, pl.ds(L, L)] = r2_v[rr, pl.ds(L, L)]
            return rc

        lax.fori_loop(0, CHUNK, body, 0)

    n2 = (w2 + CHUNK - 1) // CHUNK
    prestage(n2, p2_v, x2_v)
    pipelined(n2, emb2_hbm, r2_v, repack2, [(r2w_v, s2_hbm)])

    def repack0():
        def body(rr, rc):
            for k in range(4):
                r0a_v[rr, pl.ds(k * L, L)] = r0_v[rr, pl.ds(k * L, L)]
            for k in range(4):
                r0b_v[rr, pl.ds(k * L, L)] = r0_v[rr, pl.ds(64 + k * L, L)]
            return rc

        lax.fori_loop(0, CHUNK, body, 0)

    n0 = (w0 + CHUNK - 1) // CHUNK
    prestage(n0, p0_v, x0_v)
    pipelined(n0, emb0_hbm, r0_v, repack0,
              [(r0a_v, s1_hbm), (r0b_v, s2_hbm)])


def _sc_stage(ids, emb0, emb1, emb2):
    mesh = plsc.VectorSubcoreMesh(
        core_axis_name="c", subcore_axis_name="s",
        num_cores=NC, num_subcores=NS)
    call = pl.kernel(
        _sc_body,
        out_type=[
            jax.ShapeDtypeStruct((NSTG, 64), jnp.float32),
            jax.ShapeDtypeStruct((NSTG, 64), jnp.float32),
        ],
        mesh=mesh,
        compiler_params=pltpu.CompilerParams(
            use_tc_tiling_on_sc=False, needs_layout_passes=False),
        scratch_types=[
            pltpu.VMEM((BPW,), jnp.int32),
            pltpu.VMEM((CAP,), jnp.int32),
            pltpu.VMEM((CAP,), jnp.int32),
            pltpu.VMEM((CAP,), jnp.int32),
            pltpu.VMEM((CAP,), jnp.int32),
            pltpu.VMEM((CAP,), jnp.int32),
            pltpu.VMEM((CAP,), jnp.int32),
            pltpu.VMEM((CAP // CHUNK, CHUNK), jnp.int32),
            pltpu.VMEM((CAP // CHUNK, CHUNK), jnp.int32),
            pltpu.VMEM((CHUNK, 128), jnp.float32),
            pltpu.VMEM((CHUNK, 64), jnp.float32),
            pltpu.VMEM((CHUNK, 32), jnp.float32),
            pltpu.VMEM((CHUNK, 64), jnp.float32),
            pltpu.VMEM((CHUNK, 64), jnp.float32),
            pltpu.VMEM((CHUNK, 64), jnp.float32),
            pltpu.VMEM((CHUNK, 64), jnp.float32),
            pltpu.SemaphoreType.DMA,
            pltpu.SemaphoreType.DMA,
        ],
    )
    return call(ids, emb0, emb1, emb2)


def _tc_body(ids_ref, s1_ref, s2_ref, w1_ref, w2_ref, o_ref):
    b1 = s1_ref[...]
    b2 = s2_ref[...]
    w1t = w1_ref[...]
    w2t = w2_ref[...]
    G = B // 128
    rows = lax.broadcasted_iota(jnp.int32, (B, G), 0) // 128
    cols = lax.broadcasted_iota(jnp.int32, (B, G), 1)
    oh = (rows == cols).astype(jnp.float32)
    lane = lax.broadcasted_iota(jnp.int32, (B, 128), 1)
    pick = (lane == lax.broadcasted_iota(jnp.int32, (B, 128), 0) % 128)
    pickf = pick.astype(jnp.float32)
    outs = []
    for h in range(2):
        sl = slice(64 * h, 64 * h + 64)
        p1 = jnp.dot(b1[:, sl], w1t, preferred_element_type=jnp.float32)
        p2 = jnp.dot(b2[:, sl], w2t, preferred_element_type=jnp.float32)
        raw = jnp.concatenate([b1[:, sl], b2[:, sl]], axis=1)
        ids = ids_ref[h]
        f0 = (ids < C0_HI).astype(jnp.float32)
        f1 = (ids < C1_HI).astype(jnp.float32)
        s0 = jnp.dot(oh, f0, preferred_element_type=jnp.float32)
        s1 = jnp.dot(oh, f1, preferred_element_type=jnp.float32)
        m0 = jnp.sum(s0 * pickf, axis=1, keepdims=True) > 0.5
        m1 = jnp.sum(s1 * pickf, axis=1, keepdims=True) > 0.5
        outs.append(jnp.where(m0, raw, jnp.where(m1, p1, p2)) * SCALE)
    o_ref[...] = jnp.stack(outs, axis=0)


def _tc_combine(ids3, s1v, s2v, w1t, w2te):
    return pl.pallas_call(
        _tc_body,
        grid=(GRID,),
        in_specs=[
            pl.BlockSpec((2, B // 128, 128), lambda j: (0, j, 0)),
            pl.BlockSpec((B, 128), lambda j: (j, 0)),
            pl.BlockSpec((B, 128), lambda j: (j, 0)),
            pl.BlockSpec((64, 128), lambda j: (0, 0)),
            pl.BlockSpec((64, 128), lambda j: (0, 0)),
        ],
        out_specs=pl.BlockSpec((2, B, 128), lambda j: (0, j, 0)),
        out_shape=jax.ShapeDtypeStruct((2, NH, 128), jnp.float32),
    )(ids3, s1v, s2v, w1t, w2te)


def kernel(input_ids, emb0, emb1, emb2, W1, W2):
    ids = input_ids.T.reshape(-1)
    s1, s2 = _sc_stage(ids, emb0, emb1, emb2)
    s1v = s1.reshape(NSTG // 2, 128)
    s2v = s2.reshape(NSTG // 2, 128)
    ids3 = ids.reshape(2, NH // 128, 128)
    w2te = jnp.concatenate(
        [W2.T, jnp.zeros((32, D_OUT), jnp.float32)], axis=0)
    out = _tc_combine(ids3, s1v, s2v, W1.T, w2te)
    return out.reshape(SEQ[1], SEQ[0], D_OUT).transpose(1, 0, 2)
